# baseline (speedup 1.0000x reference)
import functools

import jax
import jax.numpy as jnp
from jax import lax
from jax.experimental import pallas as pl
from jax.experimental.pallas import tpu as pltpu
from jax.experimental.pallas import tpu_sc as plsc

_info = plsc.get_sparse_core_info()
_NC = _info.num_cores
_NS = _info.num_subcores
_NW = _NC * _NS

_CHUNK = 32
_NBUF = 3


@functools.lru_cache(maxsize=None)
def _make_gather(N, V, D):
    n_per_w = N // _NW
    n_chunks = n_per_w // _CHUNK
    mesh = plsc.VectorSubcoreMesh(core_axis_name="c", subcore_axis_name="s")

    @functools.partial(
        pl.kernel,
        mesh=mesh,
        out_type=jax.ShapeDtypeStruct((N, D), jnp.float32),
        scratch_types=[
            pltpu.VMEM((n_per_w,), jnp.int32),
            pltpu.VMEM((_NBUF, _CHUNK, D), jnp.float32),
        ] + [pltpu.SemaphoreType.DMA] * (2 * _NBUF),
    )
    def gather_kernel(table_hbm, idx_hbm, out_hbm, idx_v, rows_v, *sems):
        sem_g = sems[:_NBUF]
        sem_o = sems[_NBUF:]
        wid = lax.axis_index("s") * _NC + lax.axis_index("c")
        base = wid * n_per_w
        pltpu.sync_copy(idx_hbm.at[pl.ds(base, n_per_w)], idx_v)

        gath = [None] * _NBUF
        outc = [None] * _NBUF
        for b in range(min(_NBUF, n_chunks)):
            gath[b] = pltpu.async_copy(
                table_hbm.at[idx_v.at[pl.ds(b * _CHUNK, _CHUNK)]],
                rows_v.at[b],
                sem_g[b],
            )
        for g in range(n_chunks):
            b = g % _NBUF
            gath[b].wait()
            outc[b] = pltpu.async_copy(
                rows_v.at[b],
                out_hbm.at[pl.ds(base + g * _CHUNK, _CHUNK)],
                sem_o[b],
            )
            nxt = g + _NBUF - 1
            if g >= 1 and nxt < n_chunks:
                bp = (g - 1) % _NBUF
                outc[bp].wait()
                outc[bp] = None
                gath[bp] = pltpu.async_copy(
                    table_hbm.at[idx_v.at[pl.ds(nxt * _CHUNK, _CHUNK)]],
                    rows_v.at[bp],
                    sem_g[bp],
                )
        for b in range(_NBUF):
            if outc[b] is not None:
                outc[b].wait()

    return gather_kernel


def kernel(input_ids, table):
    B, S = input_ids.shape
    V, D = table.shape
    N = B * S
    idx_flat = input_ids.reshape(N).astype(jnp.int32)
    out = _make_gather(N, V, D)(table, idx_flat)
    return out.reshape(B, S, D)


# speedup vs baseline: 1.0126x; 1.0126x over previous
import functools

import jax
import jax.numpy as jnp
from jax import lax
from jax.experimental import pallas as pl
from jax.experimental.pallas import tpu as pltpu
from jax.experimental.pallas import tpu_sc as plsc

_info = plsc.get_sparse_core_info()
_NC = _info.num_cores
_NS = _info.num_subcores
_NW = _NC * _NS

_CHUNK = 48
_NBUF = 2


@functools.lru_cache(maxsize=None)
def _make_gather(N, V, D):
    n_per_w = N // _NW
    sizes = [_CHUNK] * (n_per_w // _CHUNK)
    if n_per_w % _CHUNK:
        sizes.append(n_per_w % _CHUNK)
    offs = [sum(sizes[:i]) for i in range(len(sizes))]
    n_chunks = len(sizes)
    mesh = plsc.VectorSubcoreMesh(core_axis_name="c", subcore_axis_name="s")

    @functools.partial(
        pl.kernel,
        mesh=mesh,
        out_type=jax.ShapeDtypeStruct((N, D), jnp.float32),
        scratch_types=[
            pltpu.VMEM((n_per_w,), jnp.int32),
            pltpu.VMEM((_NBUF, _CHUNK, D), jnp.float32),
        ] + [pltpu.SemaphoreType.DMA] * (2 * _NBUF),
    )
    def gather_kernel(table_hbm, idx_hbm, out_hbm, idx_v, rows_v, *sems):
        sem_g = sems[:_NBUF]
        sem_o = sems[_NBUF:]
        wid = lax.axis_index("s") * _NC + lax.axis_index("c")
        base = wid * n_per_w
        pltpu.sync_copy(idx_hbm.at[pl.ds(base, n_per_w)], idx_v)

        def issue_gather(c, b):
            return pltpu.async_copy(
                table_hbm.at[idx_v.at[pl.ds(offs[c], sizes[c])]],
                rows_v.at[b, pl.ds(0, sizes[c])],
                sem_g[b],
            )

        gath = [None] * _NBUF
        outc = [None] * _NBUF
        for b in range(min(_NBUF, n_chunks)):
            gath[b] = issue_gather(b, b)
        for g in range(n_chunks):
            b = g % _NBUF
            gath[b].wait()
            outc[b] = pltpu.async_copy(
                rows_v.at[b, pl.ds(0, sizes[g])],
                out_hbm.at[pl.ds(base + offs[g], sizes[g])],
                sem_o[b],
            )
            nxt = g + _NBUF
            if nxt < n_chunks:
                outc[b].wait()
                outc[b] = None
                gath[b] = issue_gather(nxt, b)
        for b in range(_NBUF):
            if outc[b] is not None:
                outc[b].wait()

    return gather_kernel


def kernel(input_ids, table):
    B, S = input_ids.shape
    V, D = table.shape
    N = B * S
    idx_flat = input_ids.reshape(N).astype(jnp.int32)
    out = _make_gather(N, V, D)(table, idx_flat)
    return out.reshape(B, S, D)


# speedup vs baseline: 1.0266x; 1.0138x over previous
import functools

import jax
import jax.numpy as jnp
from jax import lax
from jax.experimental import pallas as pl
from jax.experimental.pallas import tpu as pltpu
from jax.experimental.pallas import tpu_sc as plsc

_info = plsc.get_sparse_core_info()
_NC = _info.num_cores
_NS = _info.num_subcores
_NW = _NC * _NS

_CHUNK = 16
_NBUF = 6


@functools.lru_cache(maxsize=None)
def _make_gather(N, V, D):
    n_per_w = N // _NW
    sizes = [_CHUNK] * (n_per_w // _CHUNK)
    if n_per_w % _CHUNK:
        sizes.append(n_per_w % _CHUNK)
    offs = [sum(sizes[:i]) for i in range(len(sizes))]
    n_chunks = len(sizes)
    mesh = plsc.VectorSubcoreMesh(core_axis_name="c", subcore_axis_name="s")

    @functools.partial(
        pl.kernel,
        mesh=mesh,
        out_type=jax.ShapeDtypeStruct((N, D), jnp.float32),
        scratch_types=[
            pltpu.VMEM((n_per_w,), jnp.int32),
            pltpu.VMEM((_NBUF, _CHUNK, D), jnp.float32),
        ] + [pltpu.SemaphoreType.DMA] * (2 * _NBUF),
    )
    def gather_kernel(table_hbm, idx_hbm, out_hbm, idx_v, rows_v, *sems):
        sem_g = sems[:_NBUF]
        sem_o = sems[_NBUF:]
        wid = lax.axis_index("s") * _NC + lax.axis_index("c")
        base = wid * n_per_w
        pltpu.sync_copy(idx_hbm.at[pl.ds(base, n_per_w)], idx_v)

        def issue_gather(c, b):
            return pltpu.async_copy(
                table_hbm.at[idx_v.at[pl.ds(offs[c], sizes[c])]],
                rows_v.at[b, pl.ds(0, sizes[c])],
                sem_g[b],
            )

        gath = [None] * _NBUF
        outc = [None] * _NBUF
        for b in range(min(_NBUF, n_chunks)):
            gath[b] = issue_gather(b, b)
        for g in range(n_chunks):
            b = g % _NBUF
            gath[b].wait()
            outc[b] = pltpu.async_copy(
                rows_v.at[b, pl.ds(0, sizes[g])],
                out_hbm.at[pl.ds(base + offs[g], sizes[g])],
                sem_o[b],
            )
            nxt = g + _NBUF
            if nxt < n_chunks:
                outc[b].wait()
                outc[b] = None
                gath[b] = issue_gather(nxt, b)
        for b in range(_NBUF):
            if outc[b] is not None:
                outc[b].wait()

    return gather_kernel


def kernel(input_ids, table):
    B, S = input_ids.shape
    V, D = table.shape
    N = B * S
    idx_flat = input_ids.reshape(N).astype(jnp.int32)
    out = _make_gather(N, V, D)(table, idx_flat)
    return out.reshape(B, S, D)
